# baseline (speedup 1.0000x reference)
import functools

import jax
import jax.numpy as jnp
from jax import lax
from jax.experimental import pallas as pl
from jax.experimental.pallas import tpu as pltpu
from jax.experimental.pallas import tpu_sc as plsc

USER_N = 5000
ITEM_N = 5000
N = USER_N + ITEM_N
D = 256
E = 160000
NT = 32
HALF = N // 2
RPT = 320
LPAD = 16 * RPT
NPAD = NT * RPT
SHIFT = LPAD - HALF
DV = D // 16
PKBITS = 14
PKMASK = (1 << PKBITS) - 1
CH = 5120
EPAD = NT * CH
EB = 512
CAP = 512
SEG = NT * CAP
CAPR = CAP // 64
MROWS = 116
GB = 64
ORECIP = 6554
OSH = 21


def _partition_body(pk_hbm, w_hbm, comb_hbm, cnt_hbm,
                    pb0, wb0, recb0, wcb0, offb0, offw0,
                    pb1, wb1, recb1, wcb1, offb1, offw1,
                    cv, cvx, seml0, seml1, sems0, sems1):
    wid = lax.axis_index("s") * 2 + lax.axis_index("c")
    iota = lax.iota(jnp.int32, 16)
    zi = jnp.zeros((16,), jnp.int32)
    lane_masks = [iota == e for e in range(16)]
    NBAT = CH // EB

    slots = (
        (pb0, wb0, recb0, wcb0, offb0, offw0, seml0, sems0),
        (pb1, wb1, recb1, wcb1, offb1, offw1, seml1, sems1),
    )

    cv[pl.ds(0, 16)] = zi
    cv[pl.ds(16, 16)] = zi

    def _load(bt, s):
        pb, wb = slots[s][0], slots[s][1]
        pltpu.async_copy(pk_hbm.at[wid * NBAT + bt], pb, slots[s][6])
        pltpu.async_copy(w_hbm.at[wid * NBAT + bt], wb, slots[s][6])

    def _load_wait(bt, s):
        pb, wb = slots[s][0], slots[s][1]
        pltpu.make_async_copy(pk_hbm.at[wid * NBAT + bt], pb, slots[s][6]).wait()
        pltpu.make_async_copy(w_hbm.at[wid * NBAT + bt], wb, slots[s][6]).wait()

    def _scatter_drain(s):
        recb = slots[s][2]
        for _ in range(2 * (EB // 128)):
            pltpu.make_async_copy(recb.at[pl.ds(0, 128)],
                                  comb_hbm.at[pl.ds(0, 128)], slots[s][7]).wait()

    def _compute(s):
        pb, wb, recb, wcb, offb, offw = slots[s][:6]

        def _grp(g, _):
            sl = pl.ds(g * 16, 16)
            pv = pb[sl]
            sv = pv & PKMASK
            dv = lax.shift_right_logical(pv, PKBITS)
            sp = sv + jnp.where(sv >= HALF, SHIFT, 0)
            pd = dv + jnp.where(dv >= HALF, SHIFT, 0)
            o16 = lax.shift_right_logical(pd * ORECIP, OSH)
            lr16 = pd - o16 * RPT
            rec16 = sp | (lr16 << PKBITS)

            rankv = zi
            basev = zi
            hist_lo = zi
            hist_hi = zi
            for e in range(16):
                o_sc = o16[e]
                base_e = cv[pl.ds(o_sc, 16)][0]
                bo = o_sc + zi
                rankv = rankv + jnp.where((o16 == bo) & (iota > e), 1, 0)
                hist_lo = hist_lo + jnp.where(iota == bo, 1, 0)
                hist_hi = hist_hi + jnp.where(iota == (bo - 16), 1, 0)
                basev = jnp.where(lane_masks[e], base_e + zi, basev)
            cv[pl.ds(0, 16)] = cv[pl.ds(0, 16)] + hist_lo
            cv[pl.ds(16, 16)] = cv[pl.ds(16, 16)] + hist_hi

            pos = jnp.minimum(basev + rankv, CAP - 1)
            off16 = wid * SEG + o16 * CAP + pos
            boff = (lax.shift_right_logical(off16, 6) << 7) + (off16 & 63)
            recb[pl.ds(g * 16, 16)] = rec16.astype(jnp.float32)
            wcb[sl] = wb[sl]
            r = g >> 3
            offb[r, pl.ds((g & 7) * 16, 16)] = boff
            offw[r, pl.ds((g & 7) * 16, 16)] = boff + 64
            return 0

        lax.fori_loop(0, EB // 16, _grp, 0)

    def _scatter(s):
        recb, wcb, offb, offw = slots[s][2:6]
        for kb in range(EB // 128):
            pltpu.async_copy(recb.at[pl.ds(kb * 128, 128)],
                             comb_hbm.at[offb.at[kb]], slots[s][7])
            pltpu.async_copy(wcb.at[pl.ds(kb * 128, 128)],
                             comb_hbm.at[offw.at[kb]], slots[s][7])

    _load(0, 0)

    def _pair(i, _):
        b0 = 2 * i
        _load(b0 + 1, 1)
        _load_wait(b0, 0)

        @pl.when(i > 0)
        def _():
            _scatter_drain(0)

        _compute(0)
        _scatter(0)

        @pl.when(i < NBAT // 2 - 1)
        def _():
            _load(b0 + 2, 0)

        _load_wait(b0 + 1, 1)

        @pl.when(i > 0)
        def _():
            _scatter_drain(1)

        _compute(1)
        _scatter(1)
        return 0

    lax.fori_loop(0, NBAT // 2, _pair, 0)
    _scatter_drain(0)
    _scatter_drain(1)

    cvx[pl.ds(0, 16)] = jnp.minimum(cv[pl.ds(0, 16)], CAP)
    cvx[pl.ds(16, 16)] = jnp.minimum(cv[pl.ds(16, 16)], CAP)
    pltpu.sync_copy(cvx, cnt_hbm.at[wid])


def _layer_body(final, comb_hbm, cntt_hbm, ego_hbm, *rest):
    if final:
        (e0_hbm, e1_hbm, out_hbm, cv, cvs, frec, gidx, lrb, wvb, rows,
         acc, semr, sga, sgb) = rest
    else:
        (out_hbm, cv, cvs, frec, gidx, lrb, wvb, rows,
         acc, semr, sga, sgb) = rest

    wid = lax.axis_index("s") * 2 + lax.axis_index("c")
    iota = lax.iota(jnp.int32, 16)
    zf = jnp.zeros((16,), jnp.float32)

    pltpu.sync_copy(cntt_hbm.at[wid], cvs)
    cv[pl.ds(0, 16)] = cvs[pl.ds(0, 16)]
    cv[pl.ds(16, 16)] = cvs[pl.ds(16, 16)]

    def _seg(st, carry):
        pr, k_fly = carry
        n = cv[pl.ds(st, 16)][0]
        k64 = (n + 63) >> 6
        srow = st * (SEG * 2 // 128) + wid * (CAP * 2 // 128)

        def _sub(j, _):
            pltpu.async_copy(comb_hbm.at[srow + j], frec.at[pr + j], semr)
            return 0

        lax.fori_loop(0, k64, _sub, 0)
        k_fly = k_fly + k64
        ndrain = jnp.maximum(k_fly - 48, 0)

        def _dr(j, _):
            pltpu.make_async_copy(comb_hbm.at[0], frec.at[0], semr).wait()
            return 0

        lax.fori_loop(0, ndrain, _dr, 0)
        return (jnp.minimum(pr + k64, MROWS - CAPR), k_fly - ndrain)

    pr_end, kfly = lax.fori_loop(0, NT, _seg, (jnp.int32(0), jnp.int32(0)))

    def _zr(r, _):
        for d in range(DV):
            acc[r, pl.ds(d * 16, 16)] = zf
        return 0

    lax.fori_loop(0, RPT, _zr, 0)

    def _dr2(j, _):
        pltpu.make_async_copy(comb_hbm.at[0], frec.at[0], semr).wait()
        return 0

    lax.fori_loop(0, kfly, _dr2, 0)

    def _fix(st, pr):
        n = cv[pl.ds(st, 16)][0]
        k64 = (n + 63) >> 6
        nv = n - (k64 - 1) * 64

        @pl.when(k64 > 0)
        def _():
            row = pr + k64 - 1
            for g in range(4):
                sl = pl.ds(64 + g * 16, 16)
                valid = (g * 16 + iota) < nv
                frec[row, 0, sl] = jnp.where(valid, frec[row, 0, sl], 0.0)

        return jnp.minimum(pr + k64, MROWS - CAPR)

    lax.fori_loop(0, NT, _fix, jnp.int32(0))

    nbat = pr_end
    npair = (nbat + 1) >> 1

    def _unp(b, slot):
        def _u(g, _):
            sl = pl.ds(g * 16, 16)
            pv = frec[b, 0, sl].astype(jnp.int32)
            gidx[slot, sl] = jnp.minimum(pv & PKMASK, NPAD - 1)
            lrb[slot, sl] = jnp.minimum(
                lax.shift_right_logical(pv, PKBITS) & 511, RPT - 1)
            wvb[slot, sl] = frec[b, 0, pl.ds(64 + g * 16, 16)]
            return 0

        lax.fori_loop(0, GB // 16, _u, 0)

    def _start(slot, sem):
        pltpu.async_copy(ego_hbm.at[gidx.at[slot]], rows.at[slot], sem)

    def _gwait(slot, sem):
        pltpu.make_async_copy(ego_hbm.at[gidx.at[slot]], rows.at[slot], sem).wait()

    def _accum(slot):
        def _ag(g, _):
            lr16 = lrb[slot, pl.ds(g * 16, 16)]
            w16 = wvb[slot, pl.ds(g * 16, 16)]
            for e in range(16):
                lr = lr16[e]
                we = w16[e]
                r = g * 16 + e
                for d in range(DV):
                    sl = pl.ds(d * 16, 16)
                    acc[lr, sl] = acc[lr, sl] + rows[slot, r, sl] * we
            return 0

        lax.fori_loop(0, GB // 16, _ag, 0)

    def _bat(b, _):
        _unp(b, 0)
        _start(0, sga)
        _gwait(0, sga)
        _accum(0)
        return 0

    lax.fori_loop(0, nbat, _bat, 0)

    obase = wid * RPT
    if final:
        CK = GB
        jobs = [(other, k * CK)
                for k in range(RPT // CK)
                for other in (e0_hbm, e1_hbm, ego_hbm)]
        sems = (sga, sgb)

        def _load(j):
            other, off = jobs[j]
            pltpu.async_copy(other.at[pl.ds(obase + off, CK)],
                             rows.at[j % 2], sems[j % 2])

        _load(0)
        for j, (other, off) in enumerate(jobs):
            if j + 1 < len(jobs):
                _load(j + 1)
            pltpu.make_async_copy(other.at[pl.ds(obase + off, CK)],
                                  rows.at[j % 2], sems[j % 2]).wait()

            def _add(r, _, off=off, slot=j % 2):
                for d in range(DV):
                    sl = pl.ds(d * 16, 16)
                    acc[off + r, sl] = acc[off + r, sl] + rows[slot, r, sl]
                return 0

            lax.fori_loop(0, CK, _add, 0)

        def _scale(r, _):
            for d in range(DV):
                sl = pl.ds(d * 16, 16)
                acc[r, sl] = acc[r, sl] * 0.25
            return 0

        lax.fori_loop(0, RPT, _scale, 0)
    pltpu.sync_copy(acc, out_hbm.at[pl.ds(obase, RPT)])


_MESH = dict(core_axis_name="c", subcore_axis_name="s")


def _make_partition():
    return pl.kernel(
        _partition_body,
        out_type=(
            jax.ShapeDtypeStruct((NT * SEG * 2,), jnp.float32),
            jax.ShapeDtypeStruct((NT, NT), jnp.int32),
        ),
        mesh=plsc.VectorSubcoreMesh(**_MESH),
        scratch_types=[
            pltpu.VMEM((EB,), jnp.int32),
            pltpu.VMEM((EB,), jnp.float32),
            pltpu.VMEM((EB,), jnp.float32),
            pltpu.VMEM((EB,), jnp.float32),
            pltpu.VMEM((EB // 128, 128), jnp.int32),
            pltpu.VMEM((EB // 128, 128), jnp.int32),
            pltpu.VMEM((EB,), jnp.int32),
            pltpu.VMEM((EB,), jnp.float32),
            pltpu.VMEM((EB,), jnp.float32),
            pltpu.VMEM((EB,), jnp.float32),
            pltpu.VMEM((EB // 128, 128), jnp.int32),
            pltpu.VMEM((EB // 128, 128), jnp.int32),
            pltpu.VMEM((NT + 16,), jnp.int32),
            pltpu.VMEM((NT,), jnp.int32),
            pltpu.SemaphoreType.DMA,
            pltpu.SemaphoreType.DMA,
            pltpu.SemaphoreType.DMA,
            pltpu.SemaphoreType.DMA,
        ],
        name="lightgcn_partition",
    )


def _make_layer(final):
    return pl.kernel(
        functools.partial(_layer_body, final),
        out_type=jax.ShapeDtypeStruct((NPAD, D), jnp.float32),
        mesh=plsc.VectorSubcoreMesh(**_MESH),
        scratch_types=[
            pltpu.VMEM((NT + 16,), jnp.int32),
            pltpu.VMEM((NT,), jnp.int32),
            pltpu.VMEM((MROWS, 1, 128), jnp.float32),
            pltpu.VMEM((2, GB), jnp.int32),
            pltpu.VMEM((2, GB), jnp.int32),
            pltpu.VMEM((2, GB), jnp.float32),
            pltpu.VMEM((2, GB, D), jnp.float32),
            pltpu.VMEM((RPT, D), jnp.float32),
            pltpu.SemaphoreType.DMA,
            pltpu.SemaphoreType.DMA,
            pltpu.SemaphoreType.DMA,
        ],
        name="lightgcn_layer_final" if final else "lightgcn_layer",
    )


def kernel(edge_index, edge_weight, user_emb, item_emb):
    src = edge_index[0]
    dst = edge_index[1]
    zi = jnp.zeros((EPAD - E,), jnp.int32)
    packed = jnp.concatenate([src, zi]) | (jnp.concatenate([dst, zi]) << PKBITS)
    pk = packed.reshape(-1, EB)
    wp = jnp.concatenate([edge_weight, jnp.zeros((EPAD - E,), jnp.float32)]).reshape(-1, EB)

    ego0 = jnp.zeros((NPAD, D), jnp.float32)
    ego0 = ego0.at[:USER_N].set(user_emb).at[LPAD:LPAD + ITEM_N].set(item_emb)

    comb, cnt = _make_partition()(pk, wp)
    comb = comb.reshape(-1, 1, 128)
    cntt = cnt.T

    layer = _make_layer(False)
    layer_final = _make_layer(True)
    e1 = layer(comb, cntt, ego0)
    e2 = layer(comb, cntt, e1)
    out = layer_final(comb, cntt, e2, ego0, e1)
    return (out[:USER_N], out[LPAD:LPAD + ITEM_N])


# speedup vs baseline: 1.6379x; 1.6379x over previous
import functools

import jax
import jax.numpy as jnp
from jax import lax
from jax.experimental import pallas as pl
from jax.experimental.pallas import tpu as pltpu
from jax.experimental.pallas import tpu_sc as plsc

USER_N = 5000
ITEM_N = 5000
N = USER_N + ITEM_N
D = 256
E = 160000
NT = 32
HALF = N // 2
RPT = 320
LPAD = 16 * RPT
NPAD = NT * RPT
SHIFT = LPAD - HALF
DV = D // 16
PKBITS = 14
PKMASK = (1 << PKBITS) - 1
CH = 5120
EPAD = NT * CH
EB = 512
CAP = 448
REG = NT * CAP
GB = 64
ORECIP = 6554
OSH = 21


def _partition_body(pk_hbm, w_hbm, rec_hbm, rw_hbm, cnt_hbm, pb, wb, recb, offb, cv, cvx):
    wid = lax.axis_index("s") * 2 + lax.axis_index("c")
    iota = lax.iota(jnp.int32, 16)
    zi = jnp.zeros((16,), jnp.int32)
    lane_masks = [iota == e for e in range(16)]

    cv[pl.ds(0, 16)] = zi
    cv[pl.ds(16, 16)] = zi

    def _bt(bt, _):
        pltpu.sync_copy(pk_hbm.at[wid * (CH // EB) + bt], pb)
        pltpu.sync_copy(w_hbm.at[wid * (CH // EB) + bt], wb)

        def _grp(g, _):
            sl = pl.ds(g * 16, 16)
            pv = pb[sl]
            sv = pv & PKMASK
            dv = lax.shift_right_logical(pv, PKBITS)
            sp = sv + jnp.where(sv >= HALF, SHIFT, 0)
            pd = dv + jnp.where(dv >= HALF, SHIFT, 0)
            o16 = lax.shift_right_logical(pd * ORECIP, OSH)
            lr16 = pd - o16 * RPT
            rec16 = sp | (lr16 << PKBITS)

            rankv = zi
            basev = zi
            hist_lo = zi
            hist_hi = zi
            for e in range(16):
                o_sc = o16[e]
                base_e = cv[pl.ds(o_sc, 16)][0]
                bo = o_sc + zi
                rankv = rankv + jnp.where((o16 == bo) & (iota > e), 1, 0)
                hist_lo = hist_lo + jnp.where(iota == bo, 1, 0)
                hist_hi = hist_hi + jnp.where(iota == (bo - 16), 1, 0)
                basev = jnp.where(lane_masks[e], base_e + zi, basev)
            cv[pl.ds(0, 16)] = cv[pl.ds(0, 16)] + hist_lo
            cv[pl.ds(16, 16)] = cv[pl.ds(16, 16)] + hist_hi

            pos = jnp.minimum(basev + rankv, CAP - 1)
            off16 = o16 * REG + wid * CAP + pos
            recb[pl.ds(g * 16, 16)] = rec16
            r = g >> 3
            offb[r, pl.ds((g & 7) * 16, 16)] = off16
            return 0

        lax.fori_loop(0, EB // 16, _grp, 0)

        for kb in range(EB // 128):
            pltpu.sync_copy(recb.at[pl.ds(kb * 128, 128)], rec_hbm.at[offb.at[kb]])
            pltpu.sync_copy(wb.at[pl.ds(kb * 128, 128)], rw_hbm.at[offb.at[kb]])
        return 0

    lax.fori_loop(0, CH // EB, _bt, 0)

    cvx[pl.ds(0, 16)] = jnp.minimum(cv[pl.ds(0, 16)], CAP)
    cvx[pl.ds(16, 16)] = jnp.minimum(cv[pl.ds(16, 16)], CAP)
    pltpu.sync_copy(cvx, cnt_hbm.at[wid])


def _layer_body(final, rec_hbm, rw_hbm, cntt_hbm, ego_hbm, *rest):
    if final:
        (e0_hbm, e1_hbm, out_hbm, cv, cvs, srec, sw, gidx, lrb, wvb, rows,
         acc) = rest
    else:
        out_hbm, cv, cvs, srec, sw, gidx, lrb, wvb, rows, acc = rest

    wid = lax.axis_index("s") * 2 + lax.axis_index("c")
    iota = lax.iota(jnp.int32, 16)
    zf = jnp.zeros((16,), jnp.float32)

    pltpu.sync_copy(cntt_hbm.at[wid], cvs)
    cv[pl.ds(0, 16)] = cvs[pl.ds(0, 16)]
    cv[pl.ds(16, 16)] = cvs[pl.ds(16, 16)]
    pltpu.sync_copy(rec_hbm.at[pl.ds(wid * REG, REG)], srec)
    pltpu.sync_copy(rw_hbm.at[pl.ds(wid * REG, REG)], sw)

    def _zr(r, _):
        for d in range(DV):
            acc[r, pl.ds(d * 16, 16)] = zf
        return 0

    lax.fori_loop(0, RPT, _zr, 0)

    def _seg(st, _):
        n = cv[pl.ds(st, 16)][0]
        nb = (n + GB - 1) >> 6

        def _bat(j, _):
            base = st * CAP + j * GB

            def _u(g, _):
                sl = pl.ds(base + g * 16, 16)
                osl = pl.ds(g * 16, 16)
                pv = srec[sl]
                gidx[osl] = jnp.minimum(pv & PKMASK, NPAD - 1)
                lrb[osl] = jnp.minimum(
                    lax.shift_right_logical(pv, PKBITS) & 511, RPT - 1)
                valid = (j * GB + g * 16 + iota) < n
                wvb[osl] = jnp.where(valid, sw[sl], 0.0)
                return 0

            lax.fori_loop(0, GB // 16, _u, 0)
            pltpu.sync_copy(ego_hbm.at[gidx], rows)

            def _ag(g, _):
                lr16 = lrb[pl.ds(g * 16, 16)]
                w16 = wvb[pl.ds(g * 16, 16)]
                for e in range(16):
                    lr = lr16[e]
                    we = w16[e]
                    r = g * 16 + e
                    for d in range(DV):
                        sl = pl.ds(d * 16, 16)
                        acc[lr, sl] = acc[lr, sl] + rows[r, sl] * we
                return 0

            lax.fori_loop(0, GB // 16, _ag, 0)
            return 0

        lax.fori_loop(0, nb, _bat, 0)
        return 0

    lax.fori_loop(0, NT, _seg, 0)

    obase = wid * RPT
    if final:
        CK = GB
        for k in range(RPT // CK):
            off = k * CK
            for other in (e0_hbm, e1_hbm, ego_hbm):
                pltpu.sync_copy(other.at[pl.ds(obase + off, CK)], rows)

                def _add(r, _, off=off):
                    for d in range(DV):
                        sl = pl.ds(d * 16, 16)
                        acc[off + r, sl] = acc[off + r, sl] + rows[r, sl]
                    return 0

                lax.fori_loop(0, CK, _add, 0)

        def _scale(r, _):
            for d in range(DV):
                sl = pl.ds(d * 16, 16)
                acc[r, sl] = acc[r, sl] * 0.25
            return 0

        lax.fori_loop(0, RPT, _scale, 0)
    pltpu.sync_copy(acc, out_hbm.at[pl.ds(obase, RPT)])


_MESH = dict(core_axis_name="c", subcore_axis_name="s")


def _make_partition():
    return pl.kernel(
        _partition_body,
        out_type=(
            jax.ShapeDtypeStruct((NT * REG,), jnp.int32),
            jax.ShapeDtypeStruct((NT * REG,), jnp.float32),
            jax.ShapeDtypeStruct((NT, NT), jnp.int32),
        ),
        mesh=plsc.VectorSubcoreMesh(**_MESH),
        scratch_types=[
            pltpu.VMEM((EB,), jnp.int32),
            pltpu.VMEM((EB,), jnp.float32),
            pltpu.VMEM((EB,), jnp.int32),
            pltpu.VMEM((EB // 128, 128), jnp.int32),
            pltpu.VMEM((NT + 16,), jnp.int32),
            pltpu.VMEM((NT,), jnp.int32),
        ],
        name="lightgcn_partition",
    )


def _make_layer(final):
    return pl.kernel(
        functools.partial(_layer_body, final),
        out_type=jax.ShapeDtypeStruct((NPAD, D), jnp.float32),
        mesh=plsc.VectorSubcoreMesh(**_MESH),
        scratch_types=[
            pltpu.VMEM((NT + 16,), jnp.int32),
            pltpu.VMEM((NT,), jnp.int32),
            pltpu.VMEM((REG,), jnp.int32),
            pltpu.VMEM((REG,), jnp.float32),
            pltpu.VMEM((GB,), jnp.int32),
            pltpu.VMEM((GB,), jnp.int32),
            pltpu.VMEM((GB,), jnp.float32),
            pltpu.VMEM((GB, D), jnp.float32),
            pltpu.VMEM((RPT, D), jnp.float32),
        ],
        name="lightgcn_layer_final" if final else "lightgcn_layer",
    )


def kernel(edge_index, edge_weight, user_emb, item_emb):
    src = edge_index[0]
    dst = edge_index[1]
    zi = jnp.zeros((EPAD - E,), jnp.int32)
    packed = jnp.concatenate([src, zi]) | (jnp.concatenate([dst, zi]) << PKBITS)
    pk = packed.reshape(-1, EB)
    wp = jnp.concatenate([edge_weight, jnp.zeros((EPAD - E,), jnp.float32)]).reshape(-1, EB)

    ego0 = jnp.zeros((NPAD, D), jnp.float32)
    ego0 = ego0.at[:USER_N].set(user_emb).at[LPAD:LPAD + ITEM_N].set(item_emb)

    rec, rw, cnt = _make_partition()(pk, wp)
    cntt = cnt.T

    layer = _make_layer(False)
    layer_final = _make_layer(True)
    e1 = layer(rec, rw, cntt, ego0)
    e2 = layer(rec, rw, cntt, e1)
    out = layer_final(rec, rw, cntt, e2, ego0, e1)
    return (out[:USER_N], out[LPAD:LPAD + ITEM_N])


# speedup vs baseline: 1.8477x; 1.1281x over previous
import functools

import jax
import jax.numpy as jnp
from jax import lax
from jax.experimental import pallas as pl
from jax.experimental.pallas import tpu as pltpu
from jax.experimental.pallas import tpu_sc as plsc

USER_N = 5000
ITEM_N = 5000
N = USER_N + ITEM_N
D = 256
E = 160000
NT = 32
HALF = N // 2
RPT = 320
LPAD = 16 * RPT
NPAD = NT * RPT
SHIFT = LPAD - HALF
DV = D // 16
PKBITS = 14
PKMASK = (1 << PKBITS) - 1
CH = 5120
EPAD = NT * CH
EB = 512
CAP = 448
REG = NT * CAP
GB = 64
ORECIP = 6554
OSH = 21


def _partition_body(pk_hbm, w_hbm, rec_hbm, rw_hbm, cnt_hbm, pb, wb, recb, offb, cv, cvx):
    wid = lax.axis_index("s") * 2 + lax.axis_index("c")
    iota = lax.iota(jnp.int32, 16)
    zi = jnp.zeros((16,), jnp.int32)
    lane_masks = [iota == e for e in range(16)]

    cv[pl.ds(0, 16)] = zi
    cv[pl.ds(16, 16)] = zi

    def _bt(bt, _):
        pltpu.sync_copy(pk_hbm.at[wid * (CH // EB) + bt], pb)
        pltpu.sync_copy(w_hbm.at[wid * (CH // EB) + bt], wb)

        def _grp(g, _):
            sl = pl.ds(g * 16, 16)
            pv = pb[sl]
            sv = pv & PKMASK
            dv = lax.shift_right_logical(pv, PKBITS)
            sp = sv + jnp.where(sv >= HALF, SHIFT, 0)
            pd = dv + jnp.where(dv >= HALF, SHIFT, 0)
            o16 = lax.shift_right_logical(pd * ORECIP, OSH)
            lr16 = pd - o16 * RPT
            rec16 = sp | (lr16 << PKBITS)

            rankv = zi
            basev = zi
            hist_lo = zi
            hist_hi = zi
            for e in range(16):
                o_sc = o16[e]
                base_e = cv[pl.ds(o_sc, 16)][0]
                bo = o_sc + zi
                rankv = rankv + jnp.where((o16 == bo) & (iota > e), 1, 0)
                hist_lo = hist_lo + jnp.where(iota == bo, 1, 0)
                hist_hi = hist_hi + jnp.where(iota == (bo - 16), 1, 0)
                basev = jnp.where(lane_masks[e], base_e + zi, basev)
            cv[pl.ds(0, 16)] = cv[pl.ds(0, 16)] + hist_lo
            cv[pl.ds(16, 16)] = cv[pl.ds(16, 16)] + hist_hi

            pos = jnp.minimum(basev + rankv, CAP - 1)
            off16 = o16 * REG + wid * CAP + pos
            recb[pl.ds(g * 16, 16)] = rec16
            r = g >> 3
            offb[r, pl.ds((g & 7) * 16, 16)] = off16
            return 0

        lax.fori_loop(0, EB // 16, _grp, 0)

        for kb in range(EB // 128):
            pltpu.sync_copy(recb.at[pl.ds(kb * 128, 128)], rec_hbm.at[offb.at[kb]])
            pltpu.sync_copy(wb.at[pl.ds(kb * 128, 128)], rw_hbm.at[offb.at[kb]])
        return 0

    lax.fori_loop(0, CH // EB, _bt, 0)

    cvx[pl.ds(0, 16)] = jnp.minimum(cv[pl.ds(0, 16)], CAP)
    cvx[pl.ds(16, 16)] = jnp.minimum(cv[pl.ds(16, 16)], CAP)
    pltpu.sync_copy(cvx, cnt_hbm.at[wid])


def _layer_body(final, rec_hbm, rw_hbm, cntt_hbm, ego_hbm, *rest):
    if final:
        (e0_hbm, e1_hbm, out_hbm, cv, cvs, srec, sw, gidx, lrb, wvb, rows,
         acc) = rest
    else:
        out_hbm, cv, cvs, srec, sw, gidx, lrb, wvb, rows, acc = rest

    wid = lax.axis_index("s") * 2 + lax.axis_index("c")
    iota = lax.iota(jnp.int32, 16)
    zf = jnp.zeros((16,), jnp.float32)

    pltpu.sync_copy(cntt_hbm.at[wid], cvs)
    cv[pl.ds(0, 16)] = cvs[pl.ds(0, 16)]
    cv[pl.ds(16, 16)] = cvs[pl.ds(16, 16)]
    pltpu.sync_copy(rec_hbm.at[pl.ds(wid * REG, REG)], srec)
    pltpu.sync_copy(rw_hbm.at[pl.ds(wid * REG, REG)], sw)

    def _zr(r, _):
        for d in range(DV):
            acc[r, pl.ds(d * 16, 16)] = zf
        return 0

    lax.fori_loop(0, RPT, _zr, 0)

    def _seg(st, _):
        n = cv[pl.ds(st, 16)][0]
        nb = (n + GB - 1) >> 6

        def _bat(j, _):
            base = st * CAP + j * GB

            def _u(g, _):
                sl = pl.ds(base + g * 16, 16)
                osl = pl.ds(g * 16, 16)
                pv = srec[sl]
                gidx[osl] = jnp.minimum(pv & PKMASK, NPAD - 1)
                lrb[osl] = jnp.minimum(
                    lax.shift_right_logical(pv, PKBITS) & 511, RPT - 1)
                valid = (j * GB + g * 16 + iota) < n
                wvb[osl] = jnp.where(valid, sw[sl], 0.0)
                return 0

            lax.fori_loop(0, GB // 16, _u, 0)
            pltpu.sync_copy(ego_hbm.at[gidx], rows)

            def _ag(g, _):
                lr16 = lrb[pl.ds(g * 16, 16)]
                w16 = wvb[pl.ds(g * 16, 16)]
                for e in range(16):
                    lr = lr16[e]
                    we = w16[e]
                    r = g * 16 + e
                    for d in range(DV):
                        sl = pl.ds(d * 16, 16)
                        plsc.addupdate(acc.at[lr, sl], rows[r, sl] * we)
                return 0

            lax.fori_loop(0, GB // 16, _ag, 0)
            return 0

        lax.fori_loop(0, nb, _bat, 0)
        return 0

    lax.fori_loop(0, NT, _seg, 0)

    obase = wid * RPT
    if final:
        CK = GB
        for k in range(RPT // CK):
            off = k * CK
            for other in (e0_hbm, e1_hbm, ego_hbm):
                pltpu.sync_copy(other.at[pl.ds(obase + off, CK)], rows)

                def _add(r, _, off=off):
                    for d in range(DV):
                        sl = pl.ds(d * 16, 16)
                        plsc.addupdate(acc.at[off + r, sl], rows[r, sl])
                    return 0

                lax.fori_loop(0, CK, _add, 0)

        def _scale(r, _):
            for d in range(DV):
                sl = pl.ds(d * 16, 16)
                acc[r, sl] = acc[r, sl] * 0.25
            return 0

        lax.fori_loop(0, RPT, _scale, 0)
    pltpu.sync_copy(acc, out_hbm.at[pl.ds(obase, RPT)])


_MESH = dict(core_axis_name="c", subcore_axis_name="s")


def _make_partition():
    return pl.kernel(
        _partition_body,
        out_type=(
            jax.ShapeDtypeStruct((NT * REG,), jnp.int32),
            jax.ShapeDtypeStruct((NT * REG,), jnp.float32),
            jax.ShapeDtypeStruct((NT, NT), jnp.int32),
        ),
        mesh=plsc.VectorSubcoreMesh(**_MESH),
        scratch_types=[
            pltpu.VMEM((EB,), jnp.int32),
            pltpu.VMEM((EB,), jnp.float32),
            pltpu.VMEM((EB,), jnp.int32),
            pltpu.VMEM((EB // 128, 128), jnp.int32),
            pltpu.VMEM((NT + 16,), jnp.int32),
            pltpu.VMEM((NT,), jnp.int32),
        ],
        name="lightgcn_partition",
    )


def _make_layer(final):
    return pl.kernel(
        functools.partial(_layer_body, final),
        out_type=jax.ShapeDtypeStruct((NPAD, D), jnp.float32),
        mesh=plsc.VectorSubcoreMesh(**_MESH),
        scratch_types=[
            pltpu.VMEM((NT + 16,), jnp.int32),
            pltpu.VMEM((NT,), jnp.int32),
            pltpu.VMEM((REG,), jnp.int32),
            pltpu.VMEM((REG,), jnp.float32),
            pltpu.VMEM((GB,), jnp.int32),
            pltpu.VMEM((GB,), jnp.int32),
            pltpu.VMEM((GB,), jnp.float32),
            pltpu.VMEM((GB, D), jnp.float32),
            pltpu.VMEM((RPT, D), jnp.float32),
        ],
        name="lightgcn_layer_final" if final else "lightgcn_layer",
    )


def kernel(edge_index, edge_weight, user_emb, item_emb):
    src = edge_index[0]
    dst = edge_index[1]
    zi = jnp.zeros((EPAD - E,), jnp.int32)
    packed = jnp.concatenate([src, zi]) | (jnp.concatenate([dst, zi]) << PKBITS)
    pk = packed.reshape(-1, EB)
    wp = jnp.concatenate([edge_weight, jnp.zeros((EPAD - E,), jnp.float32)]).reshape(-1, EB)

    ego0 = jnp.zeros((NPAD, D), jnp.float32)
    ego0 = ego0.at[:USER_N].set(user_emb).at[LPAD:LPAD + ITEM_N].set(item_emb)

    rec, rw, cnt = _make_partition()(pk, wp)
    cntt = cnt.T

    layer = _make_layer(False)
    layer_final = _make_layer(True)
    e1 = layer(rec, rw, cntt, ego0)
    e2 = layer(rec, rw, cntt, e1)
    out = layer_final(rec, rw, cntt, e2, ego0, e1)
    return (out[:USER_N], out[LPAD:LPAD + ITEM_N])


# speedup vs baseline: 1.9787x; 1.0709x over previous
import functools

import jax
import jax.numpy as jnp
from jax import lax
from jax.experimental import pallas as pl
from jax.experimental.pallas import tpu as pltpu
from jax.experimental.pallas import tpu_sc as plsc

USER_N = 5000
ITEM_N = 5000
N = USER_N + ITEM_N
D = 256
E = 160000
NT = 32
HALF = N // 2
RPT = 320
LPAD = 16 * RPT
NPAD = NT * RPT
SHIFT = LPAD - HALF
DV = D // 16
PKBITS = 14
PKMASK = (1 << PKBITS) - 1
CH = 5120
EPAD = NT * CH
EB = 512
CAP = 448
REG = NT * CAP
GB = 64
ORECIP = 6554
OSH = 21


def _partition_body(pk_hbm, w_hbm, rec_hbm, rw_hbm, cnt_hbm, pb, wb, recb, offb, cv, cvx):
    wid = lax.axis_index("s") * 2 + lax.axis_index("c")
    iota = lax.iota(jnp.int32, 16)
    zi = jnp.zeros((16,), jnp.int32)
    lane_masks = [iota == e for e in range(16)]

    cv[pl.ds(0, 16)] = zi
    cv[pl.ds(16, 16)] = zi

    def _bt(bt, _):
        pltpu.sync_copy(pk_hbm.at[wid * (CH // EB) + bt], pb)
        pltpu.sync_copy(w_hbm.at[wid * (CH // EB) + bt], wb)

        def _grp(g, _):
            sl = pl.ds(g * 16, 16)
            pv = pb[sl]
            sv = pv & PKMASK
            dv = lax.shift_right_logical(pv, PKBITS)
            sp = sv + jnp.where(sv >= HALF, SHIFT, 0)
            pd = dv + jnp.where(dv >= HALF, SHIFT, 0)
            o16 = lax.shift_right_logical(pd * ORECIP, OSH)
            lr16 = pd - o16 * RPT
            rec16 = sp | (lr16 << PKBITS)

            rankv = zi
            basev = zi
            hist_lo = zi
            hist_hi = zi
            for e in range(16):
                o_sc = o16[e]
                base_e = cv[pl.ds(o_sc, 16)][0]
                bo = o_sc + zi
                rankv = rankv + jnp.where((o16 == bo) & (iota > e), 1, 0)
                hist_lo = hist_lo + jnp.where(iota == bo, 1, 0)
                hist_hi = hist_hi + jnp.where(iota == (bo - 16), 1, 0)
                basev = jnp.where(lane_masks[e], base_e + zi, basev)
            cv[pl.ds(0, 16)] = cv[pl.ds(0, 16)] + hist_lo
            cv[pl.ds(16, 16)] = cv[pl.ds(16, 16)] + hist_hi

            pos = jnp.minimum(basev + rankv, CAP - 1)
            off16 = o16 * REG + wid * CAP + pos
            recb[pl.ds(g * 16, 16)] = rec16
            r = g >> 3
            offb[r, pl.ds((g & 7) * 16, 16)] = off16
            return 0

        lax.fori_loop(0, EB // 16, _grp, 0)

        for kb in range(EB // 128):
            pltpu.sync_copy(recb.at[pl.ds(kb * 128, 128)], rec_hbm.at[offb.at[kb]])
            pltpu.sync_copy(wb.at[pl.ds(kb * 128, 128)], rw_hbm.at[offb.at[kb]])
        return 0

    lax.fori_loop(0, CH // EB, _bt, 0)

    cvx[pl.ds(0, 16)] = jnp.minimum(cv[pl.ds(0, 16)], CAP)
    cvx[pl.ds(16, 16)] = jnp.minimum(cv[pl.ds(16, 16)], CAP)
    pltpu.sync_copy(cvx, cnt_hbm.at[wid])


def _layer_body(final, rec_hbm, rw_hbm, cntt_hbm, ego_hbm, *rest):
    if final:
        (e0_hbm, e1_hbm, out_hbm, cv, cvs, srec0, sw0, srec1, sw1,
         gidx0, lrb0, wvb0, gidx1, lrb1, wvb1, rows0, rows1,
         acc, ss0, ss1, sga, sgb) = rest
    else:
        (out_hbm, cv, cvs, srec0, sw0, srec1, sw1,
         gidx0, lrb0, wvb0, gidx1, lrb1, wvb1, rows0, rows1,
         acc, ss0, ss1, sga, sgb) = rest

    wid = lax.axis_index("s") * 2 + lax.axis_index("c")
    iota = lax.iota(jnp.int32, 16)
    zf = jnp.zeros((16,), jnp.float32)
    segslots = ((srec0, sw0, ss0), (srec1, sw1, ss1))
    gslots = ((gidx0, lrb0, wvb0, rows0, sga), (gidx1, lrb1, wvb1, rows1, sgb))
    rows = rows0

    pltpu.sync_copy(cntt_hbm.at[wid], cvs)
    cv[pl.ds(0, 16)] = cvs[pl.ds(0, 16)]
    cv[pl.ds(16, 16)] = cvs[pl.ds(16, 16)]

    def _seg_load(st, ss):
        srec, sw, sem = segslots[ss]
        pltpu.async_copy(rec_hbm.at[pl.ds(wid * REG + st * CAP, CAP)], srec, sem)
        pltpu.async_copy(rw_hbm.at[pl.ds(wid * REG + st * CAP, CAP)], sw, sem)

    def _seg_wait(st, ss):
        srec, sw, sem = segslots[ss]
        pltpu.make_async_copy(rec_hbm.at[pl.ds(wid * REG + st * CAP, CAP)], srec, sem).wait()
        pltpu.make_async_copy(rw_hbm.at[pl.ds(wid * REG + st * CAP, CAP)], sw, sem).wait()

    _seg_load(0, 0)

    def _zr(r, _):
        for d in range(DV):
            acc[r, pl.ds(d * 16, 16)] = zf
        return 0

    lax.fori_loop(0, RPT, _zr, 0)

    def _proc(st, ss):
        srec, sw, _ = segslots[ss]
        n = cv[pl.ds(st, 16)][0]
        nb = (n + GB - 1) >> 6

        def _unp(j, gs):
            gidx, lrb, wvb = gslots[gs][:3]

            def _u(g, _):
                sl = pl.ds(j * GB + g * 16, 16)
                osl = pl.ds(g * 16, 16)
                pv = srec[sl]
                gidx[osl] = jnp.minimum(pv & PKMASK, NPAD - 1)
                lrb[osl] = jnp.minimum(
                    lax.shift_right_logical(pv, PKBITS) & 511, RPT - 1)
                valid = (j * GB + g * 16 + iota) < n
                wvb[osl] = jnp.where(valid, sw[sl], 0.0)
                return 0

            lax.fori_loop(0, GB // 16, _u, 0)
            pltpu.async_copy(ego_hbm.at[gidx], gslots[gs][3], gslots[gs][4])

        def _fin(gs):
            gidx, lrb, wvb, rws, sem = gslots[gs]
            pltpu.make_async_copy(ego_hbm.at[gidx], rws, sem).wait()

            def _ag(g, _):
                lr16 = lrb[pl.ds(g * 16, 16)]
                w16 = wvb[pl.ds(g * 16, 16)]
                for e in range(16):
                    lr = lr16[e]
                    we = w16[e]
                    r = g * 16 + e
                    for d in range(DV):
                        sl = pl.ds(d * 16, 16)
                        plsc.addupdate(acc.at[lr, sl], rws[r, sl] * we)
                return 0

            lax.fori_loop(0, GB // 16, _ag, 0)

        @pl.when(nb > 0)
        def _():
            _unp(0, 0)

        def _bp(p, _):
            b1 = 2 * p + 1

            @pl.when(b1 < nb)
            def _():
                _unp(b1, 1)

            _fin(0)

            @pl.when(b1 < nb)
            def _():
                @pl.when(b1 + 1 < nb)
                def _():
                    _unp(b1 + 1, 0)

                _fin(1)

            return 0

        lax.fori_loop(0, (nb + 1) >> 1, _bp, 0)

    def _segpair(i, _):
        st0 = 2 * i
        _seg_load(st0 + 1, 1)
        _seg_wait(st0, 0)
        _proc(st0, 0)

        @pl.when(i < NT // 2 - 1)
        def _():
            _seg_load(st0 + 2, 0)

        _seg_wait(st0 + 1, 1)
        _proc(st0 + 1, 1)
        return 0

    lax.fori_loop(0, NT // 2, _segpair, 0)

    obase = wid * RPT
    if final:
        CK = GB
        for k in range(RPT // CK):
            off = k * CK
            for other in (e0_hbm, e1_hbm, ego_hbm):
                pltpu.sync_copy(other.at[pl.ds(obase + off, CK)], rows)

                def _add(r, _, off=off):
                    for d in range(DV):
                        sl = pl.ds(d * 16, 16)
                        plsc.addupdate(acc.at[off + r, sl], rows[r, sl])
                    return 0

                lax.fori_loop(0, CK, _add, 0)

        def _scale(r, _):
            for d in range(DV):
                sl = pl.ds(d * 16, 16)
                acc[r, sl] = acc[r, sl] * 0.25
            return 0

        lax.fori_loop(0, RPT, _scale, 0)
    pltpu.sync_copy(acc, out_hbm.at[pl.ds(obase, RPT)])


_MESH = dict(core_axis_name="c", subcore_axis_name="s")


def _make_partition():
    return pl.kernel(
        _partition_body,
        out_type=(
            jax.ShapeDtypeStruct((NT * REG,), jnp.int32),
            jax.ShapeDtypeStruct((NT * REG,), jnp.float32),
            jax.ShapeDtypeStruct((NT, NT), jnp.int32),
        ),
        mesh=plsc.VectorSubcoreMesh(**_MESH),
        scratch_types=[
            pltpu.VMEM((EB,), jnp.int32),
            pltpu.VMEM((EB,), jnp.float32),
            pltpu.VMEM((EB,), jnp.int32),
            pltpu.VMEM((EB // 128, 128), jnp.int32),
            pltpu.VMEM((NT + 16,), jnp.int32),
            pltpu.VMEM((NT,), jnp.int32),
        ],
        name="lightgcn_partition",
    )


def _make_layer(final):
    return pl.kernel(
        functools.partial(_layer_body, final),
        out_type=jax.ShapeDtypeStruct((NPAD, D), jnp.float32),
        mesh=plsc.VectorSubcoreMesh(**_MESH),
        scratch_types=[
            pltpu.VMEM((NT + 16,), jnp.int32),
            pltpu.VMEM((NT,), jnp.int32),
            pltpu.VMEM((CAP,), jnp.int32),
            pltpu.VMEM((CAP,), jnp.float32),
            pltpu.VMEM((CAP,), jnp.int32),
            pltpu.VMEM((CAP,), jnp.float32),
            pltpu.VMEM((GB,), jnp.int32),
            pltpu.VMEM((GB,), jnp.int32),
            pltpu.VMEM((GB,), jnp.float32),
            pltpu.VMEM((GB,), jnp.int32),
            pltpu.VMEM((GB,), jnp.int32),
            pltpu.VMEM((GB,), jnp.float32),
            pltpu.VMEM((GB, D), jnp.float32),
            pltpu.VMEM((GB, D), jnp.float32),
            pltpu.VMEM((RPT, D), jnp.float32),
            pltpu.SemaphoreType.DMA,
            pltpu.SemaphoreType.DMA,
            pltpu.SemaphoreType.DMA,
            pltpu.SemaphoreType.DMA,
        ],
        name="lightgcn_layer_final" if final else "lightgcn_layer",
    )


def kernel(edge_index, edge_weight, user_emb, item_emb):
    src = edge_index[0]
    dst = edge_index[1]
    zi = jnp.zeros((EPAD - E,), jnp.int32)
    packed = jnp.concatenate([src, zi]) | (jnp.concatenate([dst, zi]) << PKBITS)
    pk = packed.reshape(-1, EB)
    wp = jnp.concatenate([edge_weight, jnp.zeros((EPAD - E,), jnp.float32)]).reshape(-1, EB)

    ego0 = jnp.zeros((NPAD, D), jnp.float32)
    ego0 = ego0.at[:USER_N].set(user_emb).at[LPAD:LPAD + ITEM_N].set(item_emb)

    rec, rw, cnt = _make_partition()(pk, wp)
    cntt = cnt.T

    layer = _make_layer(False)
    layer_final = _make_layer(True)
    e1 = layer(rec, rw, cntt, ego0)
    e2 = layer(rec, rw, cntt, e1)
    out = layer_final(rec, rw, cntt, e2, ego0, e1)
    return (out[:USER_N], out[LPAD:LPAD + ITEM_N])


# speedup vs baseline: 2.0024x; 1.0120x over previous
import functools

import jax
import jax.numpy as jnp
from jax import lax
from jax.experimental import pallas as pl
from jax.experimental.pallas import tpu as pltpu
from jax.experimental.pallas import tpu_sc as plsc

USER_N = 5000
ITEM_N = 5000
N = USER_N + ITEM_N
D = 256
E = 160000
NT = 32
HALF = N // 2
RPT = 320
LPAD = 16 * RPT
NPAD = NT * RPT
SHIFT = LPAD - HALF
DV = D // 16
PKBITS = 14
PKMASK = (1 << PKBITS) - 1
CH = 5120
EPAD = NT * CH
EB = 512
CAP = 448
REG = NT * CAP
GB = 64
ORECIP = 6554
OSH = 21


def _partition_body(pk_hbm, w_hbm, rec_hbm, rw_hbm, cnt_hbm,
                    pb0, wb0, recb0, wcb0, offb0,
                    pb1, wb1, recb1, wcb1, offb1,
                    cv, cvx, sl0, sl1, sc0, sc1):
    wid = lax.axis_index("s") * 2 + lax.axis_index("c")
    iota = lax.iota(jnp.int32, 16)
    zi = jnp.zeros((16,), jnp.int32)
    lane_masks = [iota == e for e in range(16)]
    NBAT = CH // EB
    slots = (
        (pb0, wb0, recb0, wcb0, offb0, sl0, sc0),
        (pb1, wb1, recb1, wcb1, offb1, sl1, sc1),
    )

    cv[pl.ds(0, 16)] = zi
    cv[pl.ds(16, 16)] = zi

    def _load(bt, s):
        pb, wb = slots[s][0], slots[s][1]
        pltpu.async_copy(pk_hbm.at[wid * NBAT + bt], pb, slots[s][5])
        pltpu.async_copy(w_hbm.at[wid * NBAT + bt], wb, slots[s][5])

    def _load_wait(bt, s):
        pb, wb = slots[s][0], slots[s][1]
        pltpu.make_async_copy(pk_hbm.at[wid * NBAT + bt], pb, slots[s][5]).wait()
        pltpu.make_async_copy(w_hbm.at[wid * NBAT + bt], wb, slots[s][5]).wait()

    def _scatter(s):
        recb, wcb, offb = slots[s][2], slots[s][3], slots[s][4]
        for kb in range(EB // 128):
            pltpu.async_copy(recb.at[pl.ds(kb * 128, 128)],
                             rec_hbm.at[offb.at[kb]], slots[s][6])
            pltpu.async_copy(wcb.at[pl.ds(kb * 128, 128)],
                             rw_hbm.at[offb.at[kb]], slots[s][6])

    def _scatter_drain(s):
        recb = slots[s][2]
        for _ in range(2 * (EB // 128)):
            pltpu.make_async_copy(recb.at[pl.ds(0, 128)],
                                  rec_hbm.at[pl.ds(0, 128)], slots[s][6]).wait()

    def _compute(s):
        pb, wb, recb, wcb, offb = slots[s][:5]

        def _grp(g, _):
            sl = pl.ds(g * 16, 16)
            pv = pb[sl]
            sv = pv & PKMASK
            dv = lax.shift_right_logical(pv, PKBITS)
            sp = sv + jnp.where(sv >= HALF, SHIFT, 0)
            pd = dv + jnp.where(dv >= HALF, SHIFT, 0)
            o16 = lax.shift_right_logical(pd * ORECIP, OSH)
            lr16 = pd - o16 * RPT
            rec16 = sp | (lr16 << PKBITS)

            rankv = zi
            basev = zi
            hist_lo = zi
            hist_hi = zi
            for e in range(16):
                o_sc = o16[e]
                base_e = cv[pl.ds(o_sc, 16)][0]
                bo = o_sc + zi
                rankv = rankv + jnp.where((o16 == bo) & (iota > e), 1, 0)
                hist_lo = hist_lo + jnp.where(iota == bo, 1, 0)
                hist_hi = hist_hi + jnp.where(iota == (bo - 16), 1, 0)
                basev = jnp.where(lane_masks[e], base_e + zi, basev)
            cv[pl.ds(0, 16)] = cv[pl.ds(0, 16)] + hist_lo
            cv[pl.ds(16, 16)] = cv[pl.ds(16, 16)] + hist_hi

            pos = jnp.minimum(basev + rankv, CAP - 1)
            off16 = o16 * REG + wid * CAP + pos
            recb[pl.ds(g * 16, 16)] = rec16
            wcb[pl.ds(g * 16, 16)] = wb[pl.ds(g * 16, 16)]
            r = g >> 3
            offb[r, pl.ds((g & 7) * 16, 16)] = off16
            return 0

        lax.fori_loop(0, EB // 16, _grp, 0)

    _load(0, 0)

    def _pair(i, _):
        b0 = 2 * i
        _load(b0 + 1, 1)
        _load_wait(b0, 0)

        @pl.when(i > 0)
        def _():
            _scatter_drain(0)

        _compute(0)
        _scatter(0)

        @pl.when(i < NBAT // 2 - 1)
        def _():
            _load(b0 + 2, 0)

        _load_wait(b0 + 1, 1)

        @pl.when(i > 0)
        def _():
            _scatter_drain(1)

        _compute(1)
        _scatter(1)
        return 0

    lax.fori_loop(0, NBAT // 2, _pair, 0)
    _scatter_drain(0)
    _scatter_drain(1)

    cvx[pl.ds(0, 16)] = jnp.minimum(cv[pl.ds(0, 16)], CAP)
    cvx[pl.ds(16, 16)] = jnp.minimum(cv[pl.ds(16, 16)], CAP)
    pltpu.sync_copy(cvx, cnt_hbm.at[wid])


def _layer_body(final, rec_hbm, rw_hbm, cntt_hbm, ego_hbm, *rest):
    if final:
        (e0_hbm, e1_hbm, out_hbm, cv, cvs, srec0, sw0, srec1, sw1,
         gidx0, lrb0, wvb0, gidx1, lrb1, wvb1, rows0, rows1,
         acc, ss0, ss1, sga, sgb) = rest
    else:
        (out_hbm, cv, cvs, srec0, sw0, srec1, sw1,
         gidx0, lrb0, wvb0, gidx1, lrb1, wvb1, rows0, rows1,
         acc, ss0, ss1, sga, sgb) = rest

    wid = lax.axis_index("s") * 2 + lax.axis_index("c")
    iota = lax.iota(jnp.int32, 16)
    zf = jnp.zeros((16,), jnp.float32)
    segslots = ((srec0, sw0, ss0), (srec1, sw1, ss1))
    gslots = ((gidx0, lrb0, wvb0, rows0, sga), (gidx1, lrb1, wvb1, rows1, sgb))
    rows = rows0

    pltpu.sync_copy(cntt_hbm.at[wid], cvs)
    cv[pl.ds(0, 16)] = cvs[pl.ds(0, 16)]
    cv[pl.ds(16, 16)] = cvs[pl.ds(16, 16)]

    def _seg_load(st, ss):
        srec, sw, sem = segslots[ss]
        pltpu.async_copy(rec_hbm.at[pl.ds(wid * REG + st * CAP, CAP)], srec, sem)
        pltpu.async_copy(rw_hbm.at[pl.ds(wid * REG + st * CAP, CAP)], sw, sem)

    def _seg_wait(st, ss):
        srec, sw, sem = segslots[ss]
        pltpu.make_async_copy(rec_hbm.at[pl.ds(wid * REG + st * CAP, CAP)], srec, sem).wait()
        pltpu.make_async_copy(rw_hbm.at[pl.ds(wid * REG + st * CAP, CAP)], sw, sem).wait()

    _seg_load(0, 0)

    def _zr(r, _):
        for d in range(DV):
            acc[r, pl.ds(d * 16, 16)] = zf
        return 0

    lax.fori_loop(0, RPT, _zr, 0)

    def _proc(st, ss):
        srec, sw, _ = segslots[ss]
        n = cv[pl.ds(st, 16)][0]
        nb = (n + GB - 1) >> 6

        def _unp(j, gs):
            gidx, lrb, wvb = gslots[gs][:3]

            def _u(g, _):
                sl = pl.ds(j * GB + g * 16, 16)
                osl = pl.ds(g * 16, 16)
                pv = srec[sl]
                gidx[osl] = jnp.minimum(pv & PKMASK, NPAD - 1)
                lrb[osl] = jnp.minimum(
                    lax.shift_right_logical(pv, PKBITS) & 511, RPT - 1)
                valid = (j * GB + g * 16 + iota) < n
                wvb[osl] = jnp.where(valid, sw[sl], 0.0)
                return 0

            lax.fori_loop(0, GB // 16, _u, 0)
            pltpu.async_copy(ego_hbm.at[gidx], gslots[gs][3], gslots[gs][4])

        def _fin(gs):
            gidx, lrb, wvb, rws, sem = gslots[gs]
            pltpu.make_async_copy(ego_hbm.at[gidx], rws, sem).wait()

            def _ag(g, _):
                lr16 = lrb[pl.ds(g * 16, 16)]
                w16 = wvb[pl.ds(g * 16, 16)]
                for e in range(16):
                    lr = lr16[e]
                    we = w16[e]
                    r = g * 16 + e
                    for d in range(DV):
                        sl = pl.ds(d * 16, 16)
                        plsc.addupdate(acc.at[lr, sl], rws[r, sl] * we)
                return 0

            lax.fori_loop(0, GB // 16, _ag, 0)

        @pl.when(nb > 0)
        def _():
            _unp(0, 0)

        def _bp(p, _):
            b1 = 2 * p + 1

            @pl.when(b1 < nb)
            def _():
                _unp(b1, 1)

            _fin(0)

            @pl.when(b1 < nb)
            def _():
                @pl.when(b1 + 1 < nb)
                def _():
                    _unp(b1 + 1, 0)

                _fin(1)

            return 0

        lax.fori_loop(0, (nb + 1) >> 1, _bp, 0)

    def _segpair(i, _):
        st0 = 2 * i
        _seg_load(st0 + 1, 1)
        _seg_wait(st0, 0)
        _proc(st0, 0)

        @pl.when(i < NT // 2 - 1)
        def _():
            _seg_load(st0 + 2, 0)

        _seg_wait(st0 + 1, 1)
        _proc(st0 + 1, 1)
        return 0

    lax.fori_loop(0, NT // 2, _segpair, 0)

    obase = wid * RPT
    if final:
        CK = GB
        for k in range(RPT // CK):
            off = k * CK
            for other in (e0_hbm, e1_hbm, ego_hbm):
                pltpu.sync_copy(other.at[pl.ds(obase + off, CK)], rows)

                def _add(r, _, off=off):
                    for d in range(DV):
                        sl = pl.ds(d * 16, 16)
                        plsc.addupdate(acc.at[off + r, sl], rows[r, sl])
                    return 0

                lax.fori_loop(0, CK, _add, 0)

        def _scale(r, _):
            for d in range(DV):
                sl = pl.ds(d * 16, 16)
                acc[r, sl] = acc[r, sl] * 0.25
            return 0

        lax.fori_loop(0, RPT, _scale, 0)
    pltpu.sync_copy(acc, out_hbm.at[pl.ds(obase, RPT)])


_MESH = dict(core_axis_name="c", subcore_axis_name="s")


def _make_partition():
    return pl.kernel(
        _partition_body,
        out_type=(
            jax.ShapeDtypeStruct((NT * REG,), jnp.int32),
            jax.ShapeDtypeStruct((NT * REG,), jnp.float32),
            jax.ShapeDtypeStruct((NT, NT), jnp.int32),
        ),
        mesh=plsc.VectorSubcoreMesh(**_MESH),
        scratch_types=[
            pltpu.VMEM((EB,), jnp.int32),
            pltpu.VMEM((EB,), jnp.float32),
            pltpu.VMEM((EB,), jnp.int32),
            pltpu.VMEM((EB,), jnp.float32),
            pltpu.VMEM((EB // 128, 128), jnp.int32),
            pltpu.VMEM((EB,), jnp.int32),
            pltpu.VMEM((EB,), jnp.float32),
            pltpu.VMEM((EB,), jnp.int32),
            pltpu.VMEM((EB,), jnp.float32),
            pltpu.VMEM((EB // 128, 128), jnp.int32),
            pltpu.VMEM((NT + 16,), jnp.int32),
            pltpu.VMEM((NT,), jnp.int32),
            pltpu.SemaphoreType.DMA,
            pltpu.SemaphoreType.DMA,
            pltpu.SemaphoreType.DMA,
            pltpu.SemaphoreType.DMA,
        ],
        name="lightgcn_partition",
    )


def _make_layer(final):
    return pl.kernel(
        functools.partial(_layer_body, final),
        out_type=jax.ShapeDtypeStruct((NPAD, D), jnp.float32),
        mesh=plsc.VectorSubcoreMesh(**_MESH),
        scratch_types=[
            pltpu.VMEM((NT + 16,), jnp.int32),
            pltpu.VMEM((NT,), jnp.int32),
            pltpu.VMEM((CAP,), jnp.int32),
            pltpu.VMEM((CAP,), jnp.float32),
            pltpu.VMEM((CAP,), jnp.int32),
            pltpu.VMEM((CAP,), jnp.float32),
            pltpu.VMEM((GB,), jnp.int32),
            pltpu.VMEM((GB,), jnp.int32),
            pltpu.VMEM((GB,), jnp.float32),
            pltpu.VMEM((GB,), jnp.int32),
            pltpu.VMEM((GB,), jnp.int32),
            pltpu.VMEM((GB,), jnp.float32),
            pltpu.VMEM((GB, D), jnp.float32),
            pltpu.VMEM((GB, D), jnp.float32),
            pltpu.VMEM((RPT, D), jnp.float32),
            pltpu.SemaphoreType.DMA,
            pltpu.SemaphoreType.DMA,
            pltpu.SemaphoreType.DMA,
            pltpu.SemaphoreType.DMA,
        ],
        name="lightgcn_layer_final" if final else "lightgcn_layer",
    )


def kernel(edge_index, edge_weight, user_emb, item_emb):
    src = edge_index[0]
    dst = edge_index[1]
    zi = jnp.zeros((EPAD - E,), jnp.int32)
    packed = jnp.concatenate([src, zi]) | (jnp.concatenate([dst, zi]) << PKBITS)
    pk = packed.reshape(-1, EB)
    wp = jnp.concatenate([edge_weight, jnp.zeros((EPAD - E,), jnp.float32)]).reshape(-1, EB)

    ego0 = jnp.zeros((NPAD, D), jnp.float32)
    ego0 = ego0.at[:USER_N].set(user_emb).at[LPAD:LPAD + ITEM_N].set(item_emb)

    rec, rw, cnt = _make_partition()(pk, wp)
    cntt = cnt.T

    layer = _make_layer(False)
    layer_final = _make_layer(True)
    e1 = layer(rec, rw, cntt, ego0)
    e2 = layer(rec, rw, cntt, e1)
    out = layer_final(rec, rw, cntt, e2, ego0, e1)
    return (out[:USER_N], out[LPAD:LPAD + ITEM_N])
